# Initial kernel scaffold; baseline (speedup 1.0000x reference)
import functools

import jax
import jax.numpy as jnp
from jax import lax
from jax.experimental import pallas as pl
from jax.experimental.pallas import tpu as pltpu
from jax.experimental.pallas import tpu_sc as plsc

N = 10000
E = 320000
D_IN = 128
HID = 16
HEADS = 8
D_OUT = 32

NC = 2
NS = 16
LANES = 16

EB = 128
R_TILE = 82
E_PAD = NC * NS * R_TILE * EB
N_PAD = 10016
RPT = N_PAD // NS

BN = 500
GRID_N = N // BN

_f32 = jnp.float32


def _dense1_body(x_ref, w_ref, as_ref, ad_ref, h_ref, ets_ref, etd_ref,
                 cs_ref, cd_ref):
    i = pl.program_id(0)
    h = jnp.dot(x_ref[...], w_ref[...], preferred_element_type=_f32)
    h_ref[...] = h
    ets = jnp.dot(h, as_ref[...], preferred_element_type=_f32)
    etd = jnp.dot(h, ad_ref[...], preferred_element_type=_f32)
    ets_ref[...] = ets
    etd_ref[...] = etd
    cs = jnp.max(ets, axis=0, keepdims=True)
    cd = jnp.max(etd, axis=0, keepdims=True)

    @pl.when(i == 0)
    def _():
        cs_ref[...] = cs
        cd_ref[...] = cd

    @pl.when(i > 0)
    def _():
        cs_ref[...] = jnp.maximum(cs_ref[...], cs)
        cd_ref[...] = jnp.maximum(cd_ref[...], cd)


def _make_dense1(d_h):
    return pl.pallas_call(
        _dense1_body,
        grid=(GRID_N,),
        in_specs=[
            pl.BlockSpec((BN, D_IN), lambda i: (i, 0)),
            pl.BlockSpec((D_IN, d_h), lambda i: (0, 0)),
            pl.BlockSpec((d_h, LANES), lambda i: (0, 0)),
            pl.BlockSpec((d_h, LANES), lambda i: (0, 0)),
        ],
        out_specs=[
            pl.BlockSpec((BN, d_h), lambda i: (i, 0)),
            pl.BlockSpec((BN, LANES), lambda i: (i, 0)),
            pl.BlockSpec((BN, LANES), lambda i: (i, 0)),
            pl.BlockSpec((1, LANES), lambda i: (0, 0)),
            pl.BlockSpec((1, LANES), lambda i: (0, 0)),
        ],
        out_shape=[
            jax.ShapeDtypeStruct((N, d_h), _f32),
            jax.ShapeDtypeStruct((N, LANES), _f32),
            jax.ShapeDtypeStruct((N, LANES), _f32),
            jax.ShapeDtypeStruct((1, LANES), _f32),
            jax.ShapeDtypeStruct((1, LANES), _f32),
        ],
    )


def _dense_mid_body(u0_ref, u1_ref, z0_ref, z1_ref, s_ref, b_ref, w_ref,
                    as_ref, ad_ref, h_ref, ets_ref, etd_ref, cs_ref, cd_ref):
    i = pl.program_id(0)
    rz = 1.0 / (z0_ref[...] + z1_ref[...] + 1e-16)
    rz_full = jnp.dot(rz, s_ref[...], preferred_element_type=_f32)
    x = (u0_ref[...] + u1_ref[...]) * rz_full + b_ref[...]
    x = jnp.where(x > 0, x, jnp.expm1(x))
    h = jnp.dot(x, w_ref[...], preferred_element_type=_f32)
    h_ref[...] = h
    ets = jnp.dot(h, as_ref[...], preferred_element_type=_f32)
    etd = jnp.dot(h, ad_ref[...], preferred_element_type=_f32)
    ets_ref[...] = ets
    etd_ref[...] = etd
    cs = jnp.max(ets, axis=0, keepdims=True)
    cd = jnp.max(etd, axis=0, keepdims=True)

    @pl.when(i == 0)
    def _():
        cs_ref[...] = cs
        cd_ref[...] = cd

    @pl.when(i > 0)
    def _():
        cs_ref[...] = jnp.maximum(cs_ref[...], cs)
        cd_ref[...] = jnp.maximum(cd_ref[...], cd)


def _make_dense_mid(d_u, d_h):
    return pl.pallas_call(
        _dense_mid_body,
        grid=(GRID_N,),
        in_specs=[
            pl.BlockSpec((BN, d_u), lambda i: (i, 0)),
            pl.BlockSpec((BN, d_u), lambda i: (i, 0)),
            pl.BlockSpec((BN, LANES), lambda i: (i, 0)),
            pl.BlockSpec((BN, LANES), lambda i: (i, 0)),
            pl.BlockSpec((LANES, d_u), lambda i: (0, 0)),
            pl.BlockSpec((1, d_u), lambda i: (0, 0)),
            pl.BlockSpec((d_u, d_h), lambda i: (0, 0)),
            pl.BlockSpec((d_h, LANES), lambda i: (0, 0)),
            pl.BlockSpec((d_h, LANES), lambda i: (0, 0)),
        ],
        out_specs=[
            pl.BlockSpec((BN, d_h), lambda i: (i, 0)),
            pl.BlockSpec((BN, LANES), lambda i: (i, 0)),
            pl.BlockSpec((BN, LANES), lambda i: (i, 0)),
            pl.BlockSpec((1, LANES), lambda i: (0, 0)),
            pl.BlockSpec((1, LANES), lambda i: (0, 0)),
        ],
        out_shape=[
            jax.ShapeDtypeStruct((N, d_h), _f32),
            jax.ShapeDtypeStruct((N, LANES), _f32),
            jax.ShapeDtypeStruct((N, LANES), _f32),
            jax.ShapeDtypeStruct((1, LANES), _f32),
            jax.ShapeDtypeStruct((1, LANES), _f32),
        ],
    )


def _dense_fin_body(u0_ref, u1_ref, z0_ref, z1_ref, s_ref, b_ref, out_ref):
    rz = 1.0 / (z0_ref[...] + z1_ref[...] + 1e-16)
    rz_full = jnp.dot(rz, s_ref[...], preferred_element_type=_f32)
    x = (u0_ref[...] + u1_ref[...]) * rz_full + b_ref[...]
    m = jnp.max(x, axis=-1, keepdims=True)
    ex = jnp.exp(x - m)
    lse = jnp.log(jnp.sum(ex, axis=-1, keepdims=True))
    out_ref[...] = x - m - lse


def _make_dense_fin(d_u):
    return pl.pallas_call(
        _dense_fin_body,
        grid=(GRID_N,),
        in_specs=[
            pl.BlockSpec((BN, d_u), lambda i: (i, 0)),
            pl.BlockSpec((BN, d_u), lambda i: (i, 0)),
            pl.BlockSpec((BN, LANES), lambda i: (i, 0)),
            pl.BlockSpec((BN, LANES), lambda i: (i, 0)),
            pl.BlockSpec((LANES, d_u), lambda i: (0, 0)),
            pl.BlockSpec((1, d_u), lambda i: (0, 0)),
        ],
        out_specs=pl.BlockSpec((BN, d_u), lambda i: (i, 0)),
        out_shape=jax.ShapeDtypeStruct((N, d_u), _f32),
    )


_dense1_128 = _make_dense1(HEADS * HID)
_dense_mid_128 = _make_dense_mid(HEADS * HID, HEADS * HID)
_dense_mid_32 = _make_dense_mid(HEADS * HID, D_OUT)
_dense_fin_32 = _make_dense_fin(D_OUT)


def _make_sc_attn(d_h, head_of_chunk):
    n_chunks = d_h // LANES

    def body(ets_ref, etd_ref, h_ref, src_ref, dst_ref, c_ref,
             z_out, acc_out,
             src_idx, dst_idx, es_buf, ed_buf, h_buf, p_buf, c_buf,
             z_sh, acc_sh, sem_s, sem_d, sem_h):
        ci = lax.axis_index("c")
        si = lax.axis_index("s")

        def _zero_row(r, _):
            p_buf[r, :] = jnp.zeros((LANES,), _f32)
            for k in range(n_chunks):
                h_buf[r, pl.ds(16 * k, 16)] = jnp.zeros((LANES,), _f32)
            return 0

        lax.fori_loop(0, EB, _zero_row, 0)
        row0 = si * RPT
        for k in range(RPT // EB):
            pltpu.sync_copy(h_buf, acc_sh.at[pl.ds(row0 + k * EB, EB)])
            pltpu.sync_copy(p_buf, z_sh.at[pl.ds(row0 + k * EB, EB)])
        rem = RPT % EB
        if rem:
            off = row0 + (RPT // EB) * EB
            pltpu.sync_copy(h_buf.at[pl.ds(0, rem)], acc_sh.at[pl.ds(off, rem)])
            pltpu.sync_copy(p_buf.at[pl.ds(0, rem)], z_sh.at[pl.ds(off, rem)])

        rbase = (ci * NS + si) * R_TILE
        pltpu.sync_copy(src_ref.at[pl.ds(rbase, R_TILE)], src_idx)
        pltpu.sync_copy(dst_ref.at[pl.ds(rbase, R_TILE)], dst_idx)
        pltpu.sync_copy(c_ref, c_buf)
        plsc.subcore_barrier()

        cv = c_buf[...]

        def row_body(j, _):
            sidx = src_idx.at[j]
            didx = dst_idx.at[j]
            cp1 = pltpu.async_copy(ets_ref.at[sidx], es_buf, sem_s)
            cp2 = pltpu.async_copy(etd_ref.at[didx], ed_buf, sem_d)
            cp3 = pltpu.async_copy(h_ref.at[sidx], h_buf, sem_h)
            cp1.wait()
            cp2.wait()
            cp3.wait()

            def edge_body(e, _):
                ev = es_buf[e, :] + ed_buf[e, :]
                ev = jnp.maximum(ev, 0.2 * ev)
                pv = jnp.exp(ev - cv)
                p_buf[e, :] = pv
                for k in range(n_chunks):
                    ps = p_buf[e, head_of_chunk[k]]
                    h_buf[e, pl.ds(16 * k, 16)] = h_buf[e, pl.ds(16 * k, 16)] * ps
                return 0

            lax.fori_loop(0, EB, edge_body, 0)
            pltpu.sync_copy(p_buf, z_sh.at[didx], add=True)
            pltpu.sync_copy(h_buf, acc_sh.at[didx], add=True)
            return 0

        lax.fori_loop(0, R_TILE, row_body, 0)
        plsc.subcore_barrier()

        for k in range(RPT // EB):
            off = row0 + k * EB
            pltpu.sync_copy(acc_sh.at[pl.ds(off, EB)], h_buf)
            pltpu.sync_copy(h_buf, acc_out.at[ci, pl.ds(off, EB)])
            pltpu.sync_copy(z_sh.at[pl.ds(off, EB)], p_buf)
            pltpu.sync_copy(p_buf, z_out.at[ci, pl.ds(off, EB)])
        if rem:
            off = row0 + (RPT // EB) * EB
            pltpu.sync_copy(acc_sh.at[pl.ds(off, rem)], h_buf.at[pl.ds(0, rem)])
            pltpu.sync_copy(h_buf.at[pl.ds(0, rem)], acc_out.at[ci, pl.ds(off, rem)])
            pltpu.sync_copy(z_sh.at[pl.ds(off, rem)], p_buf.at[pl.ds(0, rem)])
            pltpu.sync_copy(p_buf.at[pl.ds(0, rem)], z_out.at[ci, pl.ds(off, rem)])

    return pl.kernel(
        body,
        out_type=(
            jax.ShapeDtypeStruct((NC, N_PAD, LANES), _f32),
            jax.ShapeDtypeStruct((NC, N_PAD, d_h), _f32),
        ),
        mesh=plsc.VectorSubcoreMesh(
            core_axis_name="c", subcore_axis_name="s",
            num_cores=NC, num_subcores=NS),
        scratch_types=[
            pltpu.VMEM((R_TILE, EB), jnp.int32),
            pltpu.VMEM((R_TILE, EB), jnp.int32),
            pltpu.VMEM((EB, LANES), _f32),
            pltpu.VMEM((EB, LANES), _f32),
            pltpu.VMEM((EB, d_h), _f32),
            pltpu.VMEM((EB, LANES), _f32),
            pltpu.VMEM((LANES,), _f32),
            pltpu.VMEM_SHARED((N_PAD, LANES), _f32),
            pltpu.VMEM_SHARED((N_PAD, d_h), _f32),
            pltpu.SemaphoreType.DMA,
            pltpu.SemaphoreType.DMA,
            pltpu.SemaphoreType.DMA,
        ],
    )


_sc_attn_128 = _make_sc_attn(HEADS * HID, tuple(range(HEADS)))
_sc_attn_32 = _make_sc_attn(D_OUT, (0, 0))


def _embed_att(a):
    heads, ch = a.shape
    eye = jnp.eye(heads, dtype=_f32)
    m = (eye[:, None, :] * a[:, :, None]).reshape(heads * ch, heads)
    return jnp.pad(m, ((0, 0), (0, LANES - heads)))


def _expand_sel(heads, ch):
    s = jnp.repeat(jnp.eye(heads, dtype=_f32), ch, axis=1)
    return jnp.pad(s, ((0, LANES - heads), (0, 0)))


def _cmax(cs, cd):
    c = cs[0] + cd[0]
    return jnp.maximum(c, 0.2 * c)


def kernel(x, edge_index, W1, as1, ad1, b1, W2, as2, ad2, b2, W3, as3, ad3, b3):
    loop = jnp.arange(N, dtype=jnp.int32)
    src = jnp.concatenate([edge_index[0].astype(jnp.int32), loop])
    dst = jnp.concatenate([edge_index[1].astype(jnp.int32), loop])
    pad = E_PAD - (E + N)
    src_p = jnp.concatenate([src, jnp.zeros((pad,), jnp.int32)])
    dst_p = jnp.concatenate([dst, jnp.full((pad,), N, jnp.int32)])
    src_r = src_p.reshape(-1, EB)
    dst_r = dst_p.reshape(-1, EB)

    As1, Ad1 = _embed_att(as1), _embed_att(ad1)
    As2, Ad2 = _embed_att(as2), _embed_att(ad2)
    As3, Ad3 = _embed_att(as3), _embed_att(ad3)
    S128 = _expand_sel(HEADS, HID)
    S32 = _expand_sel(1, D_OUT)

    h1, ets1, etd1, cs1, cd1 = _dense1_128(x, W1, As1, Ad1)
    z1, a1 = _sc_attn_128(ets1, etd1, h1, src_r, dst_r, _cmax(cs1, cd1))

    h2, ets2, etd2, cs2, cd2 = _dense_mid_128(
        a1[0, :N], a1[1, :N], z1[0, :N], z1[1, :N], S128,
        b1.reshape(1, -1), W2, As2, Ad2)
    z2, a2 = _sc_attn_128(ets2, etd2, h2, src_r, dst_r, _cmax(cs2, cd2))

    h3, ets3, etd3, cs3, cd3 = _dense_mid_32(
        a2[0, :N], a2[1, :N], z2[0, :N], z2[1, :N], S128,
        b2.reshape(1, -1), W3, As3, Ad3)
    z3, a3 = _sc_attn_32(ets3, etd3, h3, src_r, dst_r, _cmax(cs3, cd3))

    return _dense_fin_32(a3[0, :N], a3[1, :N], z3[0, :N], z3[1, :N], S32,
                         b3.reshape(1, -1))


# speedup vs baseline: 27.3125x; 27.3125x over previous
import functools

import jax
import jax.numpy as jnp
from jax import lax
from jax.experimental import pallas as pl
from jax.experimental.pallas import tpu as pltpu
from jax.experimental.pallas import tpu_sc as plsc

N = 10000
E = 320000
D_IN = 128
HID = 16
HEADS = 8
D_OUT = 32

NC = 2
NS = 16
LANES = 16

EB = 128
R_TILE = 88
E_PAD = NC * NS * R_TILE * EB
N_PAD = 10112
RPT = N_PAD // NS

BN = 400
GRID_N = N // BN

_f32 = jnp.float32


def _dense1_body(x_ref, w_ref, as_ref, ad_ref, h_ref, ets_ref, etd_ref,
                 cs_ref, cd_ref):
    i = pl.program_id(0)
    h = jnp.dot(x_ref[...], w_ref[...], preferred_element_type=_f32)
    h_ref[...] = h
    ets = jnp.dot(h, as_ref[...], preferred_element_type=_f32)
    etd = jnp.dot(h, ad_ref[...], preferred_element_type=_f32)
    ets_ref[...] = ets
    etd_ref[...] = etd
    cs = jnp.max(ets, axis=0, keepdims=True)
    cd = jnp.max(etd, axis=0, keepdims=True)

    @pl.when(i == 0)
    def _():
        cs_ref[...] = cs
        cd_ref[...] = cd

    @pl.when(i > 0)
    def _():
        cs_ref[...] = jnp.maximum(cs_ref[...], cs)
        cd_ref[...] = jnp.maximum(cd_ref[...], cd)


def _make_dense1(d_h):
    return pl.pallas_call(
        _dense1_body,
        grid=(GRID_N,),
        in_specs=[
            pl.BlockSpec((BN, D_IN), lambda i: (i, 0)),
            pl.BlockSpec((D_IN, d_h), lambda i: (0, 0)),
            pl.BlockSpec((d_h, LANES), lambda i: (0, 0)),
            pl.BlockSpec((d_h, LANES), lambda i: (0, 0)),
        ],
        out_specs=[
            pl.BlockSpec((BN, d_h), lambda i: (i, 0)),
            pl.BlockSpec((BN, LANES), lambda i: (i, 0)),
            pl.BlockSpec((BN, LANES), lambda i: (i, 0)),
            pl.BlockSpec((1, LANES), lambda i: (0, 0)),
            pl.BlockSpec((1, LANES), lambda i: (0, 0)),
        ],
        out_shape=[
            jax.ShapeDtypeStruct((N, d_h), _f32),
            jax.ShapeDtypeStruct((N, LANES), _f32),
            jax.ShapeDtypeStruct((N, LANES), _f32),
            jax.ShapeDtypeStruct((1, LANES), _f32),
            jax.ShapeDtypeStruct((1, LANES), _f32),
        ],
    )


def _dense_mid_body(u0_ref, u1_ref, z0_ref, z1_ref, s_ref, b_ref, w_ref,
                    as_ref, ad_ref, h_ref, ets_ref, etd_ref, cs_ref, cd_ref):
    i = pl.program_id(0)
    rz = 1.0 / (z0_ref[...] + z1_ref[...] + 1e-16)
    rz_full = jnp.dot(rz, s_ref[...], preferred_element_type=_f32)
    x = (u0_ref[...] + u1_ref[...]) * rz_full + b_ref[...]
    x = jnp.where(x > 0, x, jnp.exp(jnp.minimum(x, 0.0)) - 1.0)
    h = jnp.dot(x, w_ref[...], preferred_element_type=_f32)
    h_ref[...] = h
    ets = jnp.dot(h, as_ref[...], preferred_element_type=_f32)
    etd = jnp.dot(h, ad_ref[...], preferred_element_type=_f32)
    ets_ref[...] = ets
    etd_ref[...] = etd
    cs = jnp.max(ets, axis=0, keepdims=True)
    cd = jnp.max(etd, axis=0, keepdims=True)

    @pl.when(i == 0)
    def _():
        cs_ref[...] = cs
        cd_ref[...] = cd

    @pl.when(i > 0)
    def _():
        cs_ref[...] = jnp.maximum(cs_ref[...], cs)
        cd_ref[...] = jnp.maximum(cd_ref[...], cd)


def _make_dense_mid(d_u, d_h):
    return pl.pallas_call(
        _dense_mid_body,
        grid=(GRID_N,),
        in_specs=[
            pl.BlockSpec((BN, d_u), lambda i: (i, 0)),
            pl.BlockSpec((BN, d_u), lambda i: (i, 0)),
            pl.BlockSpec((BN, LANES), lambda i: (i, 0)),
            pl.BlockSpec((BN, LANES), lambda i: (i, 0)),
            pl.BlockSpec((LANES, d_u), lambda i: (0, 0)),
            pl.BlockSpec((1, d_u), lambda i: (0, 0)),
            pl.BlockSpec((d_u, d_h), lambda i: (0, 0)),
            pl.BlockSpec((d_h, LANES), lambda i: (0, 0)),
            pl.BlockSpec((d_h, LANES), lambda i: (0, 0)),
        ],
        out_specs=[
            pl.BlockSpec((BN, d_h), lambda i: (i, 0)),
            pl.BlockSpec((BN, LANES), lambda i: (i, 0)),
            pl.BlockSpec((BN, LANES), lambda i: (i, 0)),
            pl.BlockSpec((1, LANES), lambda i: (0, 0)),
            pl.BlockSpec((1, LANES), lambda i: (0, 0)),
        ],
        out_shape=[
            jax.ShapeDtypeStruct((N, d_h), _f32),
            jax.ShapeDtypeStruct((N, LANES), _f32),
            jax.ShapeDtypeStruct((N, LANES), _f32),
            jax.ShapeDtypeStruct((1, LANES), _f32),
            jax.ShapeDtypeStruct((1, LANES), _f32),
        ],
    )


def _dense_fin_body(u0_ref, u1_ref, z0_ref, z1_ref, s_ref, b_ref, out_ref):
    rz = 1.0 / (z0_ref[...] + z1_ref[...] + 1e-16)
    rz_full = jnp.dot(rz, s_ref[...], preferred_element_type=_f32)
    x = (u0_ref[...] + u1_ref[...]) * rz_full + b_ref[...]
    m = jnp.max(x, axis=-1, keepdims=True)
    ex = jnp.exp(x - m)
    lse = jnp.log(jnp.sum(ex, axis=-1, keepdims=True))
    out_ref[...] = x - m - lse


def _make_dense_fin(d_u):
    return pl.pallas_call(
        _dense_fin_body,
        grid=(GRID_N,),
        in_specs=[
            pl.BlockSpec((BN, d_u), lambda i: (i, 0)),
            pl.BlockSpec((BN, d_u), lambda i: (i, 0)),
            pl.BlockSpec((BN, LANES), lambda i: (i, 0)),
            pl.BlockSpec((BN, LANES), lambda i: (i, 0)),
            pl.BlockSpec((LANES, d_u), lambda i: (0, 0)),
            pl.BlockSpec((1, d_u), lambda i: (0, 0)),
        ],
        out_specs=pl.BlockSpec((BN, d_u), lambda i: (i, 0)),
        out_shape=jax.ShapeDtypeStruct((N, d_u), _f32),
    )


_dense1_128 = _make_dense1(HEADS * HID)
_dense_mid_128 = _make_dense_mid(HEADS * HID, HEADS * HID)
_dense_mid_32 = _make_dense_mid(HEADS * HID, D_OUT)
_dense_fin_32 = _make_dense_fin(D_OUT)


def _make_sc_attn(d_h, head_of_chunk):
    n_chunks = d_h // LANES

    def body(ets_ref, etd_ref, h_ref, sd_ref, c_ref,
             z_out, acc_out,
             sd8, src8, dst8, es_buf, ed_buf, h_buf, p_buf, c_buf,
             z_sh, acc_sh, sem_s, sem_d, sem_h):
        ci = lax.axis_index("c")
        si = lax.axis_index("s")

        def _zero_row(r, _):
            p_buf[r, :] = jnp.zeros((LANES,), _f32)
            for k in range(n_chunks):
                h_buf[r, pl.ds(16 * k, 16)] = jnp.zeros((LANES,), _f32)
            return 0

        lax.fori_loop(0, EB, _zero_row, 0)
        row0 = si * RPT
        for k in range(RPT // EB):
            pltpu.sync_copy(h_buf, acc_sh.at[pl.ds(row0 + k * EB, EB)])
            pltpu.sync_copy(p_buf, z_sh.at[pl.ds(row0 + k * EB, EB)])
        rem = RPT % EB
        if rem:
            off = row0 + (RPT // EB) * EB
            pltpu.sync_copy(h_buf.at[pl.ds(0, rem)], acc_sh.at[pl.ds(off, rem)])
            pltpu.sync_copy(p_buf.at[pl.ds(0, rem)], z_sh.at[pl.ds(off, rem)])

        pltpu.sync_copy(c_ref, c_buf)
        plsc.subcore_barrier()

        cv = c_buf[...]
        rbase = (ci * NS + si) * R_TILE

        def macro_body(m, _):
            pltpu.sync_copy(sd_ref.at[pl.ds(rbase + 8 * m, 8)], sd8)

            def _unpack_row(j, _):
                for k in range(EB // LANES):
                    v = sd8[j, pl.ds(LANES * k, LANES)]
                    src8[j, pl.ds(LANES * k, LANES)] = v & 0xFFFF
                    dst8[j, pl.ds(LANES * k, LANES)] = (
                        lax.shift_right_logical(v, 16))
                return 0

            lax.fori_loop(0, 8, _unpack_row, 0)

            def row_body(j, _):
                sidx = src8.at[j]
                didx = dst8.at[j]
                cp1 = pltpu.async_copy(ets_ref.at[sidx], es_buf, sem_s)
                cp2 = pltpu.async_copy(etd_ref.at[didx], ed_buf, sem_d)
                cp3 = pltpu.async_copy(h_ref.at[sidx], h_buf, sem_h)
                cp1.wait()
                cp2.wait()
                cp3.wait()

                def edge_body(e, _):
                    ev = es_buf[e, :] + ed_buf[e, :]
                    ev = jnp.maximum(ev, 0.2 * ev)
                    pv = jnp.exp(ev - cv)
                    p_buf[e, :] = pv
                    for k in range(n_chunks):
                        ps = pv[head_of_chunk[k]]
                        h_buf[e, pl.ds(16 * k, 16)] = (
                            h_buf[e, pl.ds(16 * k, 16)] * ps)
                    return 0

                lax.fori_loop(0, EB, edge_body, 0)
                pltpu.sync_copy(p_buf, z_sh.at[didx], add=True)
                pltpu.sync_copy(h_buf, acc_sh.at[didx], add=True)
                return 0

            lax.fori_loop(0, 8, row_body, 0)
            return 0

        lax.fori_loop(0, R_TILE // 8, macro_body, 0)
        plsc.subcore_barrier()

        pltpu.sync_copy(acc_sh.at[pl.ds(row0, RPT)],
                        acc_out.at[ci, pl.ds(row0, RPT)])
        pltpu.sync_copy(z_sh.at[pl.ds(row0, RPT)],
                        z_out.at[ci, pl.ds(row0, RPT)])

    return pl.kernel(
        body,
        name=f"sc_gat_attn_{d_h}",
        out_type=(
            jax.ShapeDtypeStruct((NC, N_PAD, LANES), _f32),
            jax.ShapeDtypeStruct((NC, N_PAD, d_h), _f32),
        ),
        mesh=plsc.VectorSubcoreMesh(
            core_axis_name="c", subcore_axis_name="s",
            num_cores=NC, num_subcores=NS),
        compiler_params=pltpu.CompilerParams(use_tc_tiling_on_sc=False),
        scratch_types=[
            pltpu.VMEM((8, EB), jnp.int32),
            pltpu.VMEM((8, EB), jnp.int32),
            pltpu.VMEM((8, EB), jnp.int32),
            pltpu.VMEM((EB, LANES), _f32),
            pltpu.VMEM((EB, LANES), _f32),
            pltpu.VMEM((EB, d_h), _f32),
            pltpu.VMEM((EB, LANES), _f32),
            pltpu.VMEM((LANES,), _f32),
            pltpu.VMEM_SHARED((N_PAD, LANES), _f32),
            pltpu.VMEM_SHARED((N_PAD, d_h), _f32),
            pltpu.SemaphoreType.DMA,
            pltpu.SemaphoreType.DMA,
            pltpu.SemaphoreType.DMA,
        ],
    )


_make_sc_attn = functools.lru_cache(maxsize=None)(_make_sc_attn)


def _sc_attn_128(*args):
    return _make_sc_attn(HEADS * HID, tuple(range(HEADS)))(*args)


def _sc_attn_32(*args):
    return _make_sc_attn(D_OUT, (0, 0))(*args)


def _embed_att(a):
    heads, ch = a.shape
    eye = jnp.eye(heads, dtype=_f32)
    m = (eye[:, None, :] * a[:, :, None]).reshape(heads * ch, heads)
    return jnp.pad(m, ((0, 0), (0, LANES - heads)))


def _expand_sel(heads, ch):
    s = jnp.repeat(jnp.eye(heads, dtype=_f32), ch, axis=1)
    return jnp.pad(s, ((0, LANES - heads), (0, 0)))


def _cmax(cs, cd):
    c = cs[0] + cd[0]
    return jnp.maximum(c, 0.2 * c)


_ET_PAD_ROWS = 48000


def _et_pad(et):
    return jnp.pad(et, ((0, _ET_PAD_ROWS - N), (0, 0)))


def kernel(x, edge_index, W1, as1, ad1, b1, W2, as2, ad2, b2, W3, as3, ad3, b3):
    loop = jnp.arange(N, dtype=jnp.int32)
    src = jnp.concatenate([edge_index[0].astype(jnp.int32), loop])
    dst = jnp.concatenate([edge_index[1].astype(jnp.int32), loop])
    pad = E_PAD - (E + N)
    src_p = jnp.concatenate([src, jnp.zeros((pad,), jnp.int32)])
    dst_p = jnp.concatenate([dst, jnp.full((pad,), N, jnp.int32)])
    sd_r = (src_p | (dst_p << 16)).reshape(-1, EB)

    As1, Ad1 = _embed_att(as1), _embed_att(ad1)
    As2, Ad2 = _embed_att(as2), _embed_att(ad2)
    As3, Ad3 = _embed_att(as3), _embed_att(ad3)
    S128 = _expand_sel(HEADS, HID)
    S32 = _expand_sel(1, D_OUT)

    h1, ets1, etd1, cs1, cd1 = _dense1_128(x, W1, As1, Ad1)
    z1, a1 = _sc_attn_128(_et_pad(ets1), _et_pad(etd1), h1, sd_r, _cmax(cs1, cd1))

    h2, ets2, etd2, cs2, cd2 = _dense_mid_128(
        a1[0, :N], a1[1, :N], z1[0, :N], z1[1, :N], S128,
        b1.reshape(1, -1), W2, As2, Ad2)
    z2, a2 = _sc_attn_128(_et_pad(ets2), _et_pad(etd2), h2, sd_r, _cmax(cs2, cd2))

    h3, ets3, etd3, cs3, cd3 = _dense_mid_32(
        a2[0, :N], a2[1, :N], z2[0, :N], z2[1, :N], S128,
        b2.reshape(1, -1), W3, As3, Ad3)
    z3, a3 = _sc_attn_32(_et_pad(ets3), _et_pad(etd3), h3, sd_r, _cmax(cs3, cd3))

    return _dense_fin_32(a3[0, :N], a3[1, :N], z3[0, :N], z3[1, :N], S32,
                         b3.reshape(1, -1))


# speedup vs baseline: 65.5538x; 2.4001x over previous
import functools

import jax
import jax.numpy as jnp
from jax import lax
from jax.experimental import pallas as pl
from jax.experimental.pallas import tpu as pltpu
from jax.experimental.pallas import tpu_sc as plsc

N = 10000
E = 320000
D_IN = 128
HID = 16
HEADS = 8
D_OUT = 32

NC = 2
NS = 16
LANES = 16

EB = 128
R_TILE = 88
E_PAD = NC * NS * R_TILE * EB
N_PAD = 10112
RPT = N_PAD // NS

BN = 400
GRID_N = N // BN

_f32 = jnp.float32


def _dense1_body(x_ref, w_ref, as_ref, ad_ref, hs_ref, etd_ref,
                 cs_ref, cd_ref):
    i = pl.program_id(0)
    h = jnp.dot(x_ref[...], w_ref[...], preferred_element_type=_f32)
    ets = jnp.dot(h, as_ref[...], preferred_element_type=_f32)
    etd = jnp.dot(h, ad_ref[...], preferred_element_type=_f32)
    hs_ref[...] = jnp.concatenate([h, ets], axis=1)
    etd_ref[...] = etd
    cs = jnp.max(ets, axis=0, keepdims=True)
    cd = jnp.max(etd, axis=0, keepdims=True)

    @pl.when(i == 0)
    def _():
        cs_ref[...] = cs
        cd_ref[...] = cd

    @pl.when(i > 0)
    def _():
        cs_ref[...] = jnp.maximum(cs_ref[...], cs)
        cd_ref[...] = jnp.maximum(cd_ref[...], cd)


def _make_dense1(d_h):
    return pl.pallas_call(
        _dense1_body,
        grid=(GRID_N,),
        in_specs=[
            pl.BlockSpec((BN, D_IN), lambda i: (i, 0)),
            pl.BlockSpec((D_IN, d_h), lambda i: (0, 0)),
            pl.BlockSpec((d_h, LANES), lambda i: (0, 0)),
            pl.BlockSpec((d_h, LANES), lambda i: (0, 0)),
        ],
        out_specs=[
            pl.BlockSpec((BN, d_h + LANES), lambda i: (i, 0)),
            pl.BlockSpec((BN, LANES), lambda i: (i, 0)),
            pl.BlockSpec((1, LANES), lambda i: (0, 0)),
            pl.BlockSpec((1, LANES), lambda i: (0, 0)),
        ],
        out_shape=[
            jax.ShapeDtypeStruct((N, d_h + LANES), _f32),
            jax.ShapeDtypeStruct((N, LANES), _f32),
            jax.ShapeDtypeStruct((1, LANES), _f32),
            jax.ShapeDtypeStruct((1, LANES), _f32),
        ],
    )


def _dense_mid_body(u0_ref, u1_ref, z0_ref, z1_ref, s_ref, b_ref, w_ref,
                    as_ref, ad_ref, hs_ref, etd_ref, cs_ref, cd_ref):
    i = pl.program_id(0)
    rz = 1.0 / (z0_ref[...] + z1_ref[...] + 1e-16)
    rz_full = jnp.dot(rz, s_ref[...], preferred_element_type=_f32)
    x = (u0_ref[...] + u1_ref[...]) * rz_full + b_ref[...]
    x = jnp.where(x > 0, x, jnp.exp(jnp.minimum(x, 0.0)) - 1.0)
    h = jnp.dot(x, w_ref[...], preferred_element_type=_f32)
    ets = jnp.dot(h, as_ref[...], preferred_element_type=_f32)
    etd = jnp.dot(h, ad_ref[...], preferred_element_type=_f32)
    hs_ref[...] = jnp.concatenate([h, ets], axis=1)
    etd_ref[...] = etd
    cs = jnp.max(ets, axis=0, keepdims=True)
    cd = jnp.max(etd, axis=0, keepdims=True)

    @pl.when(i == 0)
    def _():
        cs_ref[...] = cs
        cd_ref[...] = cd

    @pl.when(i > 0)
    def _():
        cs_ref[...] = jnp.maximum(cs_ref[...], cs)
        cd_ref[...] = jnp.maximum(cd_ref[...], cd)


def _make_dense_mid(d_u, d_h):
    return pl.pallas_call(
        _dense_mid_body,
        grid=(GRID_N,),
        in_specs=[
            pl.BlockSpec((BN, d_u), lambda i: (i, 0)),
            pl.BlockSpec((BN, d_u), lambda i: (i, 0)),
            pl.BlockSpec((BN, LANES), lambda i: (i, 0)),
            pl.BlockSpec((BN, LANES), lambda i: (i, 0)),
            pl.BlockSpec((LANES, d_u), lambda i: (0, 0)),
            pl.BlockSpec((1, d_u), lambda i: (0, 0)),
            pl.BlockSpec((d_u, d_h), lambda i: (0, 0)),
            pl.BlockSpec((d_h, LANES), lambda i: (0, 0)),
            pl.BlockSpec((d_h, LANES), lambda i: (0, 0)),
        ],
        out_specs=[
            pl.BlockSpec((BN, d_h + LANES), lambda i: (i, 0)),
            pl.BlockSpec((BN, LANES), lambda i: (i, 0)),
            pl.BlockSpec((1, LANES), lambda i: (0, 0)),
            pl.BlockSpec((1, LANES), lambda i: (0, 0)),
        ],
        out_shape=[
            jax.ShapeDtypeStruct((N, d_h + LANES), _f32),
            jax.ShapeDtypeStruct((N, LANES), _f32),
            jax.ShapeDtypeStruct((1, LANES), _f32),
            jax.ShapeDtypeStruct((1, LANES), _f32),
        ],
    )


def _dense_fin_body(u0_ref, u1_ref, z0_ref, z1_ref, s_ref, b_ref, out_ref):
    rz = 1.0 / (z0_ref[...] + z1_ref[...] + 1e-16)
    rz_full = jnp.dot(rz, s_ref[...], preferred_element_type=_f32)
    x = (u0_ref[...] + u1_ref[...]) * rz_full + b_ref[...]
    m = jnp.max(x, axis=-1, keepdims=True)
    ex = jnp.exp(x - m)
    lse = jnp.log(jnp.sum(ex, axis=-1, keepdims=True))
    out_ref[...] = x - m - lse


def _make_dense_fin(d_u):
    return pl.pallas_call(
        _dense_fin_body,
        grid=(GRID_N,),
        in_specs=[
            pl.BlockSpec((BN, d_u), lambda i: (i, 0)),
            pl.BlockSpec((BN, d_u), lambda i: (i, 0)),
            pl.BlockSpec((BN, LANES), lambda i: (i, 0)),
            pl.BlockSpec((BN, LANES), lambda i: (i, 0)),
            pl.BlockSpec((LANES, d_u), lambda i: (0, 0)),
            pl.BlockSpec((1, d_u), lambda i: (0, 0)),
        ],
        out_specs=pl.BlockSpec((BN, d_u), lambda i: (i, 0)),
        out_shape=jax.ShapeDtypeStruct((N, d_u), _f32),
    )


_dense1_128 = _make_dense1(HEADS * HID)
_dense_mid_128 = _make_dense_mid(HEADS * HID, HEADS * HID)
_dense_mid_32 = _make_dense_mid(HEADS * HID, D_OUT)
_dense_fin_32 = _make_dense_fin(D_OUT)


def _make_sc_attn(d_h, head_of_chunk):
    n_chunks = d_h // LANES
    d_p = d_h + LANES

    def body(hs_ref, etd_ref, sd_ref, c_ref,
             acc_out,
             sd8, src8, dst8, ed_buf, hs_buf, c_buf,
             acc_sh, sem_s, sem_d):
        ci = lax.axis_index("c")
        si = lax.axis_index("s")

        def _zero_row(r, _):
            for k in range(n_chunks + 1):
                hs_buf[r, pl.ds(16 * k, 16)] = jnp.zeros((LANES,), _f32)
            return 0

        lax.fori_loop(0, EB, _zero_row, 0)
        row0 = si * RPT
        for k in range(RPT // EB):
            pltpu.sync_copy(hs_buf, acc_sh.at[pl.ds(row0 + k * EB, EB)])
        rem = RPT % EB
        if rem:
            off = row0 + (RPT // EB) * EB
            pltpu.sync_copy(hs_buf.at[pl.ds(0, rem)], acc_sh.at[pl.ds(off, rem)])

        pltpu.sync_copy(c_ref, c_buf)
        plsc.subcore_barrier()

        cv = c_buf[...]
        rbase = (ci * NS + si) * R_TILE

        def macro_body(m, _):
            pltpu.sync_copy(sd_ref.at[pl.ds(rbase + 8 * m, 8)], sd8)

            def _unpack_row(j, _):
                for k in range(EB // LANES):
                    v = sd8[j, pl.ds(LANES * k, LANES)]
                    src8[j, pl.ds(LANES * k, LANES)] = v & 0xFFFF
                    dst8[j, pl.ds(LANES * k, LANES)] = (
                        lax.shift_right_logical(v, 16))
                return 0

            lax.fori_loop(0, 8, _unpack_row, 0)

            def row_body(j, _):
                sidx = src8.at[j]
                didx = dst8.at[j]
                cp1 = pltpu.async_copy(hs_ref.at[sidx], hs_buf, sem_s)
                cp2 = pltpu.async_copy(etd_ref.at[didx], ed_buf, sem_d)
                cp1.wait()
                cp2.wait()

                def edge_body(e, _):
                    ev = hs_buf[e, pl.ds(d_h, LANES)] + ed_buf[e, :]
                    ev = jnp.maximum(ev, 0.2 * ev)
                    pv = jnp.exp(ev - cv)
                    hs_buf[e, pl.ds(d_h, LANES)] = pv
                    for k in range(n_chunks):
                        ps = pv[head_of_chunk[k]]
                        hs_buf[e, pl.ds(16 * k, 16)] = (
                            hs_buf[e, pl.ds(16 * k, 16)] * ps)
                    return 0

                lax.fori_loop(0, EB, edge_body, 0)
                pltpu.sync_copy(hs_buf, acc_sh.at[didx], add=True)
                return 0

            lax.fori_loop(0, 8, row_body, 0)
            return 0

        lax.fori_loop(0, R_TILE // 8, macro_body, 0)
        plsc.subcore_barrier()

        pltpu.sync_copy(acc_sh.at[pl.ds(row0, RPT)],
                        acc_out.at[ci, pl.ds(row0, RPT)])

    return pl.kernel(
        body,
        name=f"sc_gat_attn_{d_h}",
        out_type=jax.ShapeDtypeStruct((NC, N_PAD, d_p), _f32),
        mesh=plsc.VectorSubcoreMesh(
            core_axis_name="c", subcore_axis_name="s",
            num_cores=NC, num_subcores=NS),
        compiler_params=pltpu.CompilerParams(use_tc_tiling_on_sc=False),
        scratch_types=[
            pltpu.VMEM((8, EB), jnp.int32),
            pltpu.VMEM((8, EB), jnp.int32),
            pltpu.VMEM((8, EB), jnp.int32),
            pltpu.VMEM((EB, LANES), _f32),
            pltpu.VMEM((EB, d_p), _f32),
            pltpu.VMEM((LANES,), _f32),
            pltpu.VMEM_SHARED((N_PAD, d_p), _f32),
            pltpu.SemaphoreType.DMA,
            pltpu.SemaphoreType.DMA,
        ],
    )


_make_sc_attn = functools.lru_cache(maxsize=None)(_make_sc_attn)


def _sc_attn_128(*args):
    return _make_sc_attn(HEADS * HID, tuple(range(HEADS)))(*args)


def _sc_attn_32(*args):
    return _make_sc_attn(D_OUT, (0, 0))(*args)


def _embed_att(a):
    heads, ch = a.shape
    eye = jnp.eye(heads, dtype=_f32)
    m = (eye[:, None, :] * a[:, :, None]).reshape(heads * ch, heads)
    return jnp.pad(m, ((0, 0), (0, LANES - heads)))


def _expand_sel(heads, ch):
    s = jnp.repeat(jnp.eye(heads, dtype=_f32), ch, axis=1)
    return jnp.pad(s, ((0, LANES - heads), (0, 0)))


def _cmax(cs, cd):
    c = cs[0] + cd[0]
    return jnp.maximum(c, 0.2 * c)


def kernel(x, edge_index, W1, as1, ad1, b1, W2, as2, ad2, b2, W3, as3, ad3, b3):
    loop = jnp.arange(N, dtype=jnp.int32)
    src = jnp.concatenate([edge_index[0].astype(jnp.int32), loop])
    dst = jnp.concatenate([edge_index[1].astype(jnp.int32), loop])
    pad = E_PAD - (E + N)
    pad_i = jnp.arange(pad, dtype=jnp.int32)
    src_p = jnp.concatenate([src, pad_i % N])
    dst_p = jnp.concatenate([dst, N + pad_i % (N_PAD - N)])
    sd_r = (src_p | (dst_p << 16)).reshape(-1, EB)

    As1, Ad1 = _embed_att(as1), _embed_att(ad1)
    As2, Ad2 = _embed_att(as2), _embed_att(ad2)
    As3, Ad3 = _embed_att(as3), _embed_att(ad3)
    S128 = _expand_sel(HEADS, HID)
    S32 = _expand_sel(1, D_OUT)

    D1 = HEADS * HID

    hs1, etd1, cs1, cd1 = _dense1_128(x, W1, As1, Ad1)
    a1 = _sc_attn_128(hs1, etd1, sd_r, _cmax(cs1, cd1))

    hs2, etd2, cs2, cd2 = _dense_mid_128(
        a1[0, :N, :D1], a1[1, :N, :D1], a1[0, :N, D1:], a1[1, :N, D1:], S128,
        b1.reshape(1, -1), W2, As2, Ad2)
    a2 = _sc_attn_128(hs2, etd2, sd_r, _cmax(cs2, cd2))

    hs3, etd3, cs3, cd3 = _dense_mid_32(
        a2[0, :N, :D1], a2[1, :N, :D1], a2[0, :N, D1:], a2[1, :N, D1:], S128,
        b2.reshape(1, -1), W3, As3, Ad3)
    a3 = _sc_attn_32(hs3, etd3, sd_r, _cmax(cs3, cd3))

    return _dense_fin_32(a3[0, :N, :D_OUT], a3[1, :N, :D_OUT],
                         a3[0, :N, D_OUT:], a3[1, :N, D_OUT:], S32,
                         b3.reshape(1, -1))


# speedup vs baseline: 83.0119x; 1.2663x over previous
import functools

import jax
import jax.numpy as jnp
from jax import lax
from jax.experimental import pallas as pl
from jax.experimental.pallas import tpu as pltpu
from jax.experimental.pallas import tpu_sc as plsc

N = 10000
E = 320000
D_IN = 128
HID = 16
HEADS = 8
D_OUT = 32

NC = 2
NS = 16
LANES = 16

EB = 128
R_TILE = 88
E_PAD = NC * NS * R_TILE * EB
N_PAD = 10112
RPT = N_PAD // NS

BN = 400
GRID_N = N // BN

_f32 = jnp.float32


def _dense1_body(x_ref, w_ref, as_ref, ad_ref, hs_ref, etd_ref,
                 cs_ref, cd_ref):
    i = pl.program_id(0)
    h = jnp.dot(x_ref[...], w_ref[...], preferred_element_type=_f32)
    ets = jnp.dot(h, as_ref[...], preferred_element_type=_f32)
    etd = jnp.dot(h, ad_ref[...], preferred_element_type=_f32)
    hs_ref[...] = jnp.concatenate([h, ets], axis=1)
    etd_ref[...] = etd
    cs = jnp.max(ets, axis=0, keepdims=True)
    cd = jnp.max(etd, axis=0, keepdims=True)

    @pl.when(i == 0)
    def _():
        cs_ref[...] = cs
        cd_ref[...] = cd

    @pl.when(i > 0)
    def _():
        cs_ref[...] = jnp.maximum(cs_ref[...], cs)
        cd_ref[...] = jnp.maximum(cd_ref[...], cd)


def _make_dense1(d_h):
    return pl.pallas_call(
        _dense1_body,
        grid=(GRID_N,),
        in_specs=[
            pl.BlockSpec((BN, D_IN), lambda i: (i, 0)),
            pl.BlockSpec((D_IN, d_h), lambda i: (0, 0)),
            pl.BlockSpec((d_h, LANES), lambda i: (0, 0)),
            pl.BlockSpec((d_h, LANES), lambda i: (0, 0)),
        ],
        out_specs=[
            pl.BlockSpec((BN, d_h + LANES), lambda i: (i, 0)),
            pl.BlockSpec((BN, LANES), lambda i: (i, 0)),
            pl.BlockSpec((1, LANES), lambda i: (0, 0)),
            pl.BlockSpec((1, LANES), lambda i: (0, 0)),
        ],
        out_shape=[
            jax.ShapeDtypeStruct((N, d_h + LANES), _f32),
            jax.ShapeDtypeStruct((N, LANES), _f32),
            jax.ShapeDtypeStruct((1, LANES), _f32),
            jax.ShapeDtypeStruct((1, LANES), _f32),
        ],
    )


def _dense_mid_body(u0_ref, u1_ref, z0_ref, z1_ref, s_ref, b_ref, w_ref,
                    as_ref, ad_ref, hs_ref, etd_ref, cs_ref, cd_ref):
    i = pl.program_id(0)
    rz = 1.0 / (z0_ref[...] + z1_ref[...] + 1e-16)
    rz_full = jnp.dot(rz, s_ref[...], preferred_element_type=_f32)
    x = (u0_ref[...] + u1_ref[...]) * rz_full + b_ref[...]
    x = jnp.where(x > 0, x, jnp.exp(jnp.minimum(x, 0.0)) - 1.0)
    h = jnp.dot(x, w_ref[...], preferred_element_type=_f32)
    ets = jnp.dot(h, as_ref[...], preferred_element_type=_f32)
    etd = jnp.dot(h, ad_ref[...], preferred_element_type=_f32)
    hs_ref[...] = jnp.concatenate([h, ets], axis=1)
    etd_ref[...] = etd
    cs = jnp.max(ets, axis=0, keepdims=True)
    cd = jnp.max(etd, axis=0, keepdims=True)

    @pl.when(i == 0)
    def _():
        cs_ref[...] = cs
        cd_ref[...] = cd

    @pl.when(i > 0)
    def _():
        cs_ref[...] = jnp.maximum(cs_ref[...], cs)
        cd_ref[...] = jnp.maximum(cd_ref[...], cd)


def _make_dense_mid(d_u, d_h):
    return pl.pallas_call(
        _dense_mid_body,
        grid=(GRID_N,),
        in_specs=[
            pl.BlockSpec((BN, d_u), lambda i: (i, 0)),
            pl.BlockSpec((BN, d_u), lambda i: (i, 0)),
            pl.BlockSpec((BN, LANES), lambda i: (i, 0)),
            pl.BlockSpec((BN, LANES), lambda i: (i, 0)),
            pl.BlockSpec((LANES, d_u), lambda i: (0, 0)),
            pl.BlockSpec((1, d_u), lambda i: (0, 0)),
            pl.BlockSpec((d_u, d_h), lambda i: (0, 0)),
            pl.BlockSpec((d_h, LANES), lambda i: (0, 0)),
            pl.BlockSpec((d_h, LANES), lambda i: (0, 0)),
        ],
        out_specs=[
            pl.BlockSpec((BN, d_h + LANES), lambda i: (i, 0)),
            pl.BlockSpec((BN, LANES), lambda i: (i, 0)),
            pl.BlockSpec((1, LANES), lambda i: (0, 0)),
            pl.BlockSpec((1, LANES), lambda i: (0, 0)),
        ],
        out_shape=[
            jax.ShapeDtypeStruct((N, d_h + LANES), _f32),
            jax.ShapeDtypeStruct((N, LANES), _f32),
            jax.ShapeDtypeStruct((1, LANES), _f32),
            jax.ShapeDtypeStruct((1, LANES), _f32),
        ],
    )


def _dense_fin_body(u0_ref, u1_ref, z0_ref, z1_ref, s_ref, b_ref, out_ref):
    rz = 1.0 / (z0_ref[...] + z1_ref[...] + 1e-16)
    rz_full = jnp.dot(rz, s_ref[...], preferred_element_type=_f32)
    x = (u0_ref[...] + u1_ref[...]) * rz_full + b_ref[...]
    m = jnp.max(x, axis=-1, keepdims=True)
    ex = jnp.exp(x - m)
    lse = jnp.log(jnp.sum(ex, axis=-1, keepdims=True))
    out_ref[...] = x - m - lse


def _make_dense_fin(d_u):
    return pl.pallas_call(
        _dense_fin_body,
        grid=(GRID_N,),
        in_specs=[
            pl.BlockSpec((BN, d_u), lambda i: (i, 0)),
            pl.BlockSpec((BN, d_u), lambda i: (i, 0)),
            pl.BlockSpec((BN, LANES), lambda i: (i, 0)),
            pl.BlockSpec((BN, LANES), lambda i: (i, 0)),
            pl.BlockSpec((LANES, d_u), lambda i: (0, 0)),
            pl.BlockSpec((1, d_u), lambda i: (0, 0)),
        ],
        out_specs=pl.BlockSpec((BN, d_u), lambda i: (i, 0)),
        out_shape=jax.ShapeDtypeStruct((N, d_u), _f32),
    )


_dense1_128 = _make_dense1(HEADS * HID)
_dense_mid_128 = _make_dense_mid(HEADS * HID, HEADS * HID)
_dense_mid_32 = _make_dense_mid(HEADS * HID, D_OUT)
_dense_fin_32 = _make_dense_fin(D_OUT)


def _make_sc_attn(d_h, head_of_chunk):
    n_chunks = d_h // LANES
    d_p = d_h + LANES
    HB = EB // 2
    NB = R_TILE * 2

    def body(hs_ref, etd_ref, sd_ref, c_ref,
             acc_out,
             sd8, src16, dst16, ed_buf, hs_buf, c_buf,
             acc_sh, sem_g0, sem_g1, sem_e0, sem_e1, sem_s0, sem_s1):
        ci = lax.axis_index("c")
        si = lax.axis_index("s")

        def _zero_row(r, _):
            for k in range(n_chunks + 1):
                hs_buf[r, pl.ds(16 * k, 16)] = jnp.zeros((LANES,), _f32)
            return 0

        lax.fori_loop(0, EB, _zero_row, 0)
        row0 = si * RPT
        for k in range(RPT // EB):
            pltpu.sync_copy(hs_buf, acc_sh.at[pl.ds(row0 + k * EB, EB)])
        rem = RPT % EB
        if rem:
            off = row0 + (RPT // EB) * EB
            pltpu.sync_copy(hs_buf.at[pl.ds(0, rem)], acc_sh.at[pl.ds(off, rem)])

        pltpu.sync_copy(c_ref, c_buf)
        plsc.subcore_barrier()

        cv = c_buf[...]
        rbase = (ci * NS + si) * R_TILE

        hs_half = (hs_buf.at[pl.ds(0, HB)], hs_buf.at[pl.ds(HB, HB)])
        ed_half = (ed_buf.at[pl.ds(0, HB)], ed_buf.at[pl.ds(HB, HB)])
        sem_g = (sem_g0, sem_g1)
        sem_e = (sem_e0, sem_e1)
        sem_s = (sem_s0, sem_s1)

        def load_macro(m):
            bank = m % 2
            pltpu.sync_copy(sd_ref.at[pl.ds(rbase + 8 * m, 8)], sd8.at[bank])

            def _unpack_row(j, _):
                for k in range(EB // LANES):
                    v = sd8[bank, j, pl.ds(LANES * k, LANES)]
                    src16[bank, 2 * j + k // 4, pl.ds(LANES * (k % 4), LANES)] = (
                        v & 0xFFFF)
                    dst16[bank, 2 * j + k // 4, pl.ds(LANES * (k % 4), LANES)] = (
                        lax.shift_right_logical(v, 16))
                return 0

            lax.fori_loop(0, 8, _unpack_row, 0)

        def _sidx(b):
            return src16.at[(b // 16) % 2, b % 16]

        def _didx(b):
            return dst16.at[(b // 16) % 2, b % 16]

        def issue_gather(b, h):
            pltpu.async_copy(hs_ref.at[_sidx(b)], hs_half[h], sem_g[h])
            pltpu.async_copy(etd_ref.at[_didx(b)], ed_half[h], sem_e[h])

        def wait_gather(b, h):
            pltpu.make_async_copy(hs_ref.at[_sidx(b)], hs_half[h],
                                  sem_g[h]).wait()
            pltpu.make_async_copy(etd_ref.at[_didx(b)], ed_half[h],
                                  sem_e[h]).wait()

        def issue_scatter(b, h):
            pltpu.async_copy(hs_half[h], acc_sh.at[_didx(b)], sem_s[h],
                             add=True)

        def wait_scatter(b, h):
            pltpu.make_async_copy(hs_half[h], acc_sh.at[_didx(b)],
                                  sem_s[h]).wait()

        def compute(h):
            base = HB * h

            def edge_body(e, _):
                ev = hs_buf[base + e, pl.ds(d_h, LANES)] + ed_buf[base + e, :]
                ev = jnp.maximum(ev, 0.2 * ev)
                pv = jnp.exp(ev - cv)
                hs_buf[base + e, pl.ds(d_h, LANES)] = pv
                for k in range(n_chunks):
                    ps = pv[head_of_chunk[k]]
                    hs_buf[base + e, pl.ds(16 * k, 16)] = (
                        hs_buf[base + e, pl.ds(16 * k, 16)] * ps)
                return 0

            lax.fori_loop(0, HB, edge_body, 0)

        load_macro(0)
        issue_gather(0, 0)

        def pair_body(p, _):
            b0 = 2 * p
            b1 = b0 + 1
            wait_gather(b0, 0)

            @pl.when(p > 0)
            def _():
                wait_scatter(b0 - 1, 1)

            @pl.when(jnp.logical_and(b0 % 16 == 0, b0 + 16 < NB))
            def _():
                load_macro(b0 // 16 + 1)

            issue_gather(b1, 1)
            compute(0)
            issue_scatter(b0, 0)
            wait_gather(b1, 1)
            wait_scatter(b0, 0)

            @pl.when(b1 + 1 < NB)
            def _():
                issue_gather(b1 + 1, 0)

            compute(1)
            issue_scatter(b1, 1)
            return 0

        lax.fori_loop(0, NB // 2, pair_body, 0)
        wait_scatter(NB - 1, 1)
        plsc.subcore_barrier()

        pltpu.sync_copy(acc_sh.at[pl.ds(row0, RPT)],
                        acc_out.at[ci, pl.ds(row0, RPT)])

    return pl.kernel(
        body,
        name=f"sc_gat_attn_{d_h}",
        out_type=jax.ShapeDtypeStruct((NC, N_PAD, d_p), _f32),
        mesh=plsc.VectorSubcoreMesh(
            core_axis_name="c", subcore_axis_name="s",
            num_cores=NC, num_subcores=NS),
        compiler_params=pltpu.CompilerParams(use_tc_tiling_on_sc=False),
        scratch_types=[
            pltpu.VMEM((2, 8, EB), jnp.int32),
            pltpu.VMEM((2, 16, EB // 2), jnp.int32),
            pltpu.VMEM((2, 16, EB // 2), jnp.int32),
            pltpu.VMEM((EB, LANES), _f32),
            pltpu.VMEM((EB, d_p), _f32),
            pltpu.VMEM((LANES,), _f32),
            pltpu.VMEM_SHARED((N_PAD, d_p), _f32),
            pltpu.SemaphoreType.DMA,
            pltpu.SemaphoreType.DMA,
            pltpu.SemaphoreType.DMA,
            pltpu.SemaphoreType.DMA,
            pltpu.SemaphoreType.DMA,
            pltpu.SemaphoreType.DMA,
        ],
    )


_make_sc_attn = functools.lru_cache(maxsize=None)(_make_sc_attn)


def _sc_attn_128(*args):
    return _make_sc_attn(HEADS * HID, tuple(range(HEADS)))(*args)


def _sc_attn_32(*args):
    return _make_sc_attn(D_OUT, (0, 0))(*args)


def _embed_att(a):
    heads, ch = a.shape
    eye = jnp.eye(heads, dtype=_f32)
    m = (eye[:, None, :] * a[:, :, None]).reshape(heads * ch, heads)
    return jnp.pad(m, ((0, 0), (0, LANES - heads)))


def _expand_sel(heads, ch):
    s = jnp.repeat(jnp.eye(heads, dtype=_f32), ch, axis=1)
    return jnp.pad(s, ((0, LANES - heads), (0, 0)))


def _cmax(cs, cd):
    c = cs[0] + cd[0]
    return jnp.maximum(c, 0.2 * c)


def kernel(x, edge_index, W1, as1, ad1, b1, W2, as2, ad2, b2, W3, as3, ad3, b3):
    loop = jnp.arange(N, dtype=jnp.int32)
    src = jnp.concatenate([edge_index[0].astype(jnp.int32), loop])
    dst = jnp.concatenate([edge_index[1].astype(jnp.int32), loop])
    pad = E_PAD - (E + N)
    pad_i = jnp.arange(pad, dtype=jnp.int32)
    src_p = jnp.concatenate([src, pad_i % N])
    dst_p = jnp.concatenate([dst, N + pad_i % (N_PAD - N)])
    sd_r = (src_p | (dst_p << 16)).reshape(-1, EB)

    As1, Ad1 = _embed_att(as1), _embed_att(ad1)
    As2, Ad2 = _embed_att(as2), _embed_att(ad2)
    As3, Ad3 = _embed_att(as3), _embed_att(ad3)
    S128 = _expand_sel(HEADS, HID)
    S32 = _expand_sel(1, D_OUT)

    D1 = HEADS * HID

    hs1, etd1, cs1, cd1 = _dense1_128(x, W1, As1, Ad1)
    a1 = _sc_attn_128(hs1, etd1, sd_r, _cmax(cs1, cd1))

    hs2, etd2, cs2, cd2 = _dense_mid_128(
        a1[0, :N, :D1], a1[1, :N, :D1], a1[0, :N, D1:], a1[1, :N, D1:], S128,
        b1.reshape(1, -1), W2, As2, Ad2)
    a2 = _sc_attn_128(hs2, etd2, sd_r, _cmax(cs2, cd2))

    hs3, etd3, cs3, cd3 = _dense_mid_32(
        a2[0, :N, :D1], a2[1, :N, :D1], a2[0, :N, D1:], a2[1, :N, D1:], S128,
        b2.reshape(1, -1), W3, As3, Ad3)
    a3 = _sc_attn_32(hs3, etd3, sd_r, _cmax(cs3, cd3))

    return _dense_fin_32(a3[0, :N, :D_OUT], a3[1, :N, :D_OUT],
                         a3[0, :N, D_OUT:], a3[1, :N, D_OUT:], S32,
                         b3.reshape(1, -1))
